# baseline (speedup 1.0000x reference)
import functools

import jax
import jax.numpy as jnp
from jax import lax
from jax.experimental import pallas as pl
from jax.experimental.pallas import tpu as pltpu
from jax.experimental.pallas import tpu_sc as plsc

N = 10000
D = 128
E = 320000

NC = 2
NS = 16
NW = NC * NS
CH = 80
CPT = 128
BLK = 8
NBLK = CPT // BLK
EPT = CPT * CH
EPAD = NW * EPT
NPAD = 10240
TRASH = N
RPT = NPAD // NS


RING = 4
AHEAD = RING - 1


@functools.lru_cache(maxsize=None)
def _make_spmm(F: int):
    mesh = plsc.VectorSubcoreMesh(core_axis_name="c", subcore_axis_name="s")

    scratch = (
        [pltpu.VMEM((CH, F), jnp.float32) for _ in range(RING)]
        + [pltpu.VMEM((BLK, CH), jnp.int32) for _ in range(2)]
        + [pltpu.VMEM((BLK, CH), jnp.int32) for _ in range(2)]
        + [pltpu.VMEM_SHARED((NPAD, F), jnp.float32)]
        + [pltpu.SemaphoreType.DMA for _ in range(2 * RING + 4)]
    )

    @functools.partial(
        pl.kernel,
        out_type=jax.ShapeDtypeStruct((NC, NPAD, F), jnp.float32),
        mesh=mesh,
        scratch_types=scratch,
    )
    def spmm(support_hbm, srcb_hbm, dstb_hbm, zeros_hbm, out_hbm, *scr):
        rows = scr[0:RING]
        srcB = scr[RING:RING + 2]
        dstB = scr[RING + 2:RING + 4]
        acc_sh = scr[RING + 4]
        g_sem = scr[RING + 5:RING + 5 + RING]
        s_sem = scr[RING + 5 + RING:RING + 5 + 2 * RING]
        iS_sem = scr[RING + 5 + 2 * RING:RING + 5 + 2 * RING + 2]
        iD_sem = scr[RING + 5 + 2 * RING + 2:RING + 5 + 2 * RING + 4]

        cid = lax.axis_index("c")
        sid = lax.axis_index("s")
        wid = sid * NC + cid

        pltpu.sync_copy(zeros_hbm.at[pl.ds(sid * RPT, RPT)],
                        acc_sh.at[pl.ds(sid * RPT, RPT)])
        plsc.subcore_barrier()

        def idx_start(jb, buf):
            pltpu.async_copy(srcb_hbm.at[wid, pl.ds(jb * BLK, BLK)],
                             srcB[buf], iS_sem[buf])
            pltpu.async_copy(dstb_hbm.at[wid, pl.ds(jb * BLK, BLK)],
                             dstB[buf], iD_sem[buf])

        def idx_wait(jb, buf):
            pltpu.make_async_copy(srcb_hbm.at[wid, pl.ds(jb * BLK, BLK)],
                                  srcB[buf], iS_sem[buf]).wait()
            pltpu.make_async_copy(dstb_hbm.at[wid, pl.ds(jb * BLK, BLK)],
                                  dstB[buf], iD_sem[buf]).wait()

        def gather_start(slot, buf, rb):
            pltpu.async_copy(support_hbm.at[srcB[buf].at[slot]],
                             rows[rb], g_sem[rb])

        def gather_wait(slot, buf, rb):
            pltpu.make_async_copy(support_hbm.at[srcB[buf].at[slot]],
                                  rows[rb], g_sem[rb]).wait()

        def scatter_start(slot, buf, rb):
            pltpu.async_copy(rows[rb], acc_sh.at[dstB[buf].at[slot]],
                             s_sem[rb], add=True)

        def scatter_drain(slot, buf, rb):
            pltpu.make_async_copy(rows[rb], acc_sh.at[dstB[buf].at[slot]],
                                  s_sem[rb]).wait()

        idx_start(0, 0)
        idx_wait(0, 0)
        for c in range(AHEAD):
            gather_start(c, 0, c % RING)

        def do_block(jb, buf):
            nbuf = 1 - buf
            for k in range(BLK):
                g = jb * BLK + k
                c_slot = (k + AHEAD) % BLK
                c_rb = (k + AHEAD) % RING
                c_buf = buf if k < BLK - AHEAD else nbuf

                if k == 1:
                    @pl.when(jb < NBLK - 1)
                    def _load():
                        idx_start(jb + 1, nbuf)

                if k == 0:
                    @pl.when(jb > 0)
                    def _drain0():
                        scatter_drain(BLK - 1, nbuf, c_rb)
                else:
                    scatter_drain(k - 1, buf, c_rb)

                if k == BLK - AHEAD:
                    @pl.when(jb < NBLK - 1)
                    def _wait_idx():
                        idx_wait(jb + 1, nbuf)
                if k < BLK - AHEAD:
                    gather_start(c_slot, c_buf, c_rb)
                else:
                    @pl.when(jb < NBLK - 1)
                    def _pref():
                        gather_start(c_slot, c_buf, c_rb)

                gather_wait(k, buf, k % RING)
                scatter_start(k, buf, k % RING)

        def body(jj, carry):
            do_block(2 * jj, 0)
            do_block(2 * jj + 1, 1)
            return carry

        lax.fori_loop(0, NBLK // 2, body, 0)

        scatter_drain(BLK - 1, (NBLK - 1) % 2, (CPT - 1) % RING)
        plsc.subcore_barrier()

        pltpu.sync_copy(acc_sh.at[pl.ds(sid * RPT, RPT)],
                        out_hbm.at[cid, pl.ds(sid * RPT, RPT)])

    return spmm


def _spmm(support, srcb, dstb, zeros):
    return _make_spmm(support.shape[1])(support, srcb, dstb, zeros)


BM = 2000


def _mm_body(x_ref, w_ref, o_ref):
    o_ref[...] = jnp.dot(x_ref[...], w_ref[...],
                         preferred_element_type=jnp.float32)


def _mm(x, w):
    m, k = x.shape
    f = w.shape[1]
    return pl.pallas_call(
        _mm_body,
        grid=(m // BM,),
        in_specs=[pl.BlockSpec((BM, k), lambda i: (i, 0)),
                  pl.BlockSpec((k, f), lambda i: (0, 0))],
        out_specs=pl.BlockSpec((BM, f), lambda i: (i, 0)),
        out_shape=jax.ShapeDtypeStruct((m, f), jnp.float32),
    )(x, w)


def _relu_mm_body(p_ref, b_ref, w_ref, o_ref):
    h = jnp.maximum(p_ref[0] + p_ref[1] + b_ref[...], 0.0)
    o_ref[...] = jnp.dot(h, w_ref[...], preferred_element_type=jnp.float32)


def _relu_mm(p, b, w):
    f = p.shape[2]
    f2 = w.shape[1]
    return pl.pallas_call(
        _relu_mm_body,
        grid=(N // BM,),
        in_specs=[pl.BlockSpec((NC, BM, f), lambda i: (0, i, 0)),
                  pl.BlockSpec((1, f), lambda i: (0, 0)),
                  pl.BlockSpec((f, f2), lambda i: (0, 0))],
        out_specs=pl.BlockSpec((BM, f2), lambda i: (i, 0)),
        out_shape=jax.ShapeDtypeStruct((N, f2), jnp.float32),
    )(p, b.reshape(1, f), w)


def _normalize_rows(v, eps=1e-12):
    n = jnp.sqrt(jnp.sum(v * v, axis=1, keepdims=True))
    return v / jnp.maximum(n, eps)


def _norm_body(p_ref, b_ref, o_ref):
    o_ref[...] = _normalize_rows(p_ref[0] + p_ref[1] + b_ref[...])


def _bias_normalize(p, b):
    f = p.shape[2]
    return pl.pallas_call(
        _norm_body,
        grid=(N // BM,),
        in_specs=[pl.BlockSpec((NC, BM, f), lambda i: (0, i, 0)),
                  pl.BlockSpec((1, f), lambda i: (0, 0))],
        out_specs=pl.BlockSpec((BM, f), lambda i: (i, 0)),
        out_shape=jax.ShapeDtypeStruct((N, f), jnp.float32),
    )(p, b.reshape(1, f))


def _final_body(x_ref, q0_ref, b0_ref, q1_ref, b1_ref, o_ref):
    x = x_ref[...]
    s0 = _normalize_rows(q0_ref[0] + q0_ref[1] + b0_ref[...])
    s1 = _normalize_rows(q1_ref[0] + q1_ref[1] + b1_ref[...])
    c1 = _normalize_rows(jnp.concatenate([x, s0], axis=1))
    o_ref[...] = _normalize_rows(jnp.concatenate([c1, s1], axis=1))


def _final(x, q0, b0, q1, b1):
    f = D
    return pl.pallas_call(
        _final_body,
        grid=(N // BM,),
        in_specs=[pl.BlockSpec((BM, f), lambda i: (i, 0)),
                  pl.BlockSpec((NC, BM, f), lambda i: (0, i, 0)),
                  pl.BlockSpec((1, f), lambda i: (0, 0)),
                  pl.BlockSpec((NC, BM, f), lambda i: (0, i, 0)),
                  pl.BlockSpec((1, f), lambda i: (0, 0))],
        out_specs=pl.BlockSpec((BM, 3 * f), lambda i: (i, 0)),
        out_shape=jax.ShapeDtypeStruct((N, 3 * f), jnp.float32),
    )(x, q0, b0.reshape(1, f), q1, b1.reshape(1, f))


def kernel(x, edge_index, W1_00, b1_00, W2_00, b2_00, W1_10, b1_10, W2_10,
           b2_10, W1_11, b1_11, W2_11, b2_11):
    pad = EPAD - E
    srcb = jnp.concatenate(
        [edge_index[0], jnp.zeros((pad,), jnp.int32)]).reshape(NW, CPT, CH)
    dstb = jnp.concatenate(
        [edge_index[1], jnp.full((pad,), TRASH, jnp.int32)]).reshape(NW, CPT, CH)
    zeros = jnp.zeros((NPAD, D), jnp.float32)

    def gcbs(h, W1, b1, W2):
        t = _mm(h, W1)
        a = _spmm(t, srcb, dstb, zeros)
        t2 = _relu_mm(a, b1, W2)
        return _spmm(t2, srcb, dstb, zeros)

    q00 = gcbs(x, W1_00, b1_00, W2_00)
    q10 = gcbs(x, W1_10, b1_10, W2_10)
    s10 = _bias_normalize(q10, b2_10)
    q11 = gcbs(s10, W1_11, b1_11, W2_11)

    return _final(x, q00, b2_00, q11, b2_11)


# speedup vs baseline: 5.0017x; 5.0017x over previous
import functools

import jax
import jax.numpy as jnp
from jax import lax
from jax.experimental import pallas as pl
from jax.experimental.pallas import tpu as pltpu
from jax.experimental.pallas import tpu_sc as plsc

N = 10000
D = 128
E = 320000

NC = 2
NS = 16
NW = NC * NS
EPW = E // NW
CH = 80
NCHUNK = EPW // CH
NPAD = 10240
RPT = NPAD // NS


RING = 3
AHEAD = RING - 1
NITER = -(-NCHUNK // RING)
LANES = 16
SHIFT = 14


@functools.lru_cache(maxsize=None)
def _make_spmm(F: int):
    mesh = plsc.VectorSubcoreMesh(core_axis_name="c", subcore_axis_name="s")

    scratch = (
        [pltpu.VMEM((EPW,), jnp.int32)]
        + [pltpu.VMEM((CH,), jnp.int32) for _ in range(RING)]
        + [pltpu.VMEM((CH,), jnp.int32) for _ in range(RING)]
        + [pltpu.VMEM((CH, F), jnp.float32) for _ in range(RING)]
        + [pltpu.VMEM_SHARED((NPAD, F), jnp.float32)]
        + [pltpu.SemaphoreType.DMA for _ in range(2 * RING)]
    )

    @functools.partial(
        pl.kernel,
        out_type=jax.ShapeDtypeStruct((NC, NPAD, F), jnp.float32),
        mesh=mesh,
        scratch_types=scratch,
    )
    def spmm(support_hbm, edges_hbm, zeros_hbm, out_hbm, *scr):
        idx_all = scr[0]
        src_v = scr[1:1 + RING]
        dst_v = scr[1 + RING:1 + 2 * RING]
        rows = scr[1 + 2 * RING:1 + 3 * RING]
        acc_sh = scr[1 + 3 * RING]
        g_sem = scr[2 + 3 * RING:2 + 4 * RING]
        s_sem = scr[2 + 4 * RING:2 + 5 * RING]

        cid = lax.axis_index("c")
        sid = lax.axis_index("s")
        wid = sid * NC + cid

        pltpu.async_copy(edges_hbm.at[pl.ds(wid * EPW, EPW)], idx_all,
                         g_sem[0])
        pltpu.sync_copy(zeros_hbm.at[pl.ds(sid * RPT, RPT)],
                        acc_sh.at[pl.ds(sid * RPT, RPT)])
        pltpu.make_async_copy(edges_hbm.at[pl.ds(wid * EPW, EPW)], idx_all,
                              g_sem[0]).wait()
        plsc.subcore_barrier()

        def unpack_and_gather(c, b):
            base = pl.multiple_of(c * CH, LANES)
            for i in range(CH // LANES):
                p = idx_all[pl.ds(base + i * LANES, LANES)]
                src_v[b][pl.ds(i * LANES, LANES)] = p & ((1 << SHIFT) - 1)
                dst_v[b][pl.ds(i * LANES, LANES)] = lax.shift_right_logical(
                    p, SHIFT)
            pltpu.async_copy(support_hbm.at[src_v[b]], rows[b], g_sem[b])

        for c in range(AHEAD):
            unpack_and_gather(c, c)

        def body(j, carry):
            for k in range(RING):
                g = j * RING + k
                bc = (k + AHEAD) % RING
                c = g + AHEAD

                @pl.when(c < NCHUNK)
                def _prefetch():
                    @pl.when(c >= RING)
                    def _drain():
                        pltpu.make_async_copy(
                            rows[bc], acc_sh.at[dst_v[bc]], s_sem[bc]).wait()
                    unpack_and_gather(c, bc)

                @pl.when(g < NCHUNK)
                def _consume():
                    pltpu.make_async_copy(
                        support_hbm.at[src_v[k]], rows[k], g_sem[k]).wait()
                    pltpu.async_copy(rows[k], acc_sh.at[dst_v[k]], s_sem[k],
                                     add=True)
            return carry

        lax.fori_loop(0, NITER, body, 0)

        for q in range(NCHUNK - RING, NCHUNK):
            b = q % RING
            pltpu.make_async_copy(rows[b], acc_sh.at[dst_v[b]], s_sem[b]).wait()
        plsc.subcore_barrier()

        pltpu.sync_copy(acc_sh.at[pl.ds(sid * RPT, RPT)],
                        out_hbm.at[cid, pl.ds(sid * RPT, RPT)])

    return spmm


def _spmm(support, edges, zeros):
    return _make_spmm(support.shape[1])(support, edges, zeros)


BM = 2000


def _mm_body(x_ref, w_ref, o_ref):
    o_ref[...] = jnp.dot(x_ref[...], w_ref[...],
                         preferred_element_type=jnp.float32)


def _mm(x, w):
    m, k = x.shape
    f = w.shape[1]
    return pl.pallas_call(
        _mm_body,
        grid=(m // BM,),
        in_specs=[pl.BlockSpec((BM, k), lambda i: (i, 0)),
                  pl.BlockSpec((k, f), lambda i: (0, 0))],
        out_specs=pl.BlockSpec((BM, f), lambda i: (i, 0)),
        out_shape=jax.ShapeDtypeStruct((m, f), jnp.float32),
    )(x, w)


def _relu_mm_body(p_ref, b_ref, w_ref, o_ref):
    h = jnp.maximum(p_ref[0] + p_ref[1] + b_ref[...], 0.0)
    o_ref[...] = jnp.dot(h, w_ref[...], preferred_element_type=jnp.float32)


def _relu_mm(p, b, w):
    f = p.shape[2]
    f2 = w.shape[1]
    return pl.pallas_call(
        _relu_mm_body,
        grid=(N // BM,),
        in_specs=[pl.BlockSpec((NC, BM, f), lambda i: (0, i, 0)),
                  pl.BlockSpec((1, f), lambda i: (0, 0)),
                  pl.BlockSpec((f, f2), lambda i: (0, 0))],
        out_specs=pl.BlockSpec((BM, f2), lambda i: (i, 0)),
        out_shape=jax.ShapeDtypeStruct((N, f2), jnp.float32),
    )(p, b.reshape(1, f), w)


def _normalize_rows(v, eps=1e-12):
    n = jnp.sqrt(jnp.sum(v * v, axis=1, keepdims=True))
    return v / jnp.maximum(n, eps)


def _norm_body(p_ref, b_ref, o_ref):
    o_ref[...] = _normalize_rows(p_ref[0] + p_ref[1] + b_ref[...])


def _bias_normalize(p, b):
    f = p.shape[2]
    return pl.pallas_call(
        _norm_body,
        grid=(N // BM,),
        in_specs=[pl.BlockSpec((NC, BM, f), lambda i: (0, i, 0)),
                  pl.BlockSpec((1, f), lambda i: (0, 0))],
        out_specs=pl.BlockSpec((BM, f), lambda i: (i, 0)),
        out_shape=jax.ShapeDtypeStruct((N, f), jnp.float32),
    )(p, b.reshape(1, f))


def _final_body(x_ref, q0_ref, b0_ref, q1_ref, b1_ref, o_ref):
    x = x_ref[...]
    s0 = _normalize_rows(q0_ref[0] + q0_ref[1] + b0_ref[...])
    s1 = _normalize_rows(q1_ref[0] + q1_ref[1] + b1_ref[...])
    c1 = _normalize_rows(jnp.concatenate([x, s0], axis=1))
    o_ref[...] = _normalize_rows(jnp.concatenate([c1, s1], axis=1))


def _final(x, q0, b0, q1, b1):
    f = D
    return pl.pallas_call(
        _final_body,
        grid=(N // BM,),
        in_specs=[pl.BlockSpec((BM, f), lambda i: (i, 0)),
                  pl.BlockSpec((NC, BM, f), lambda i: (0, i, 0)),
                  pl.BlockSpec((1, f), lambda i: (0, 0)),
                  pl.BlockSpec((NC, BM, f), lambda i: (0, i, 0)),
                  pl.BlockSpec((1, f), lambda i: (0, 0))],
        out_specs=pl.BlockSpec((BM, 3 * f), lambda i: (i, 0)),
        out_shape=jax.ShapeDtypeStruct((N, 3 * f), jnp.float32),
    )(x, q0, b0.reshape(1, f), q1, b1.reshape(1, f))


def kernel(x, edge_index, W1_00, b1_00, W2_00, b2_00, W1_10, b1_10, W2_10,
           b2_10, W1_11, b1_11, W2_11, b2_11):
    edges = edge_index[1] * (1 << SHIFT) + edge_index[0]
    zeros = jnp.zeros((NPAD, D), jnp.float32)

    def gcbs(h, W1, b1, W2):
        t = _mm(h, W1)
        a = _spmm(t, edges, zeros)
        t2 = _relu_mm(a, b1, W2)
        return _spmm(t2, edges, zeros)

    q00 = gcbs(x, W1_00, b1_00, W2_00)
    q10 = gcbs(x, W1_10, b1_10, W2_10)
    s10 = _bias_normalize(q10, b2_10)
    q11 = gcbs(s10, W1_11, b1_11, W2_11)

    return _final(x, q00, b2_00, q11, b2_11)
